# baseline (speedup 1.0000x reference)
import functools

import jax
import jax.numpy as jnp
from jax import lax
from jax.experimental import pallas as pl
from jax.experimental.pallas import tpu as pltpu
from jax.experimental.pallas import tpu_sc as plsc

_EMB = 32
_B = 4096
_L = 200
_NW = 32
_VPAD = 1000064
_NBLK = _VPAD // 128
_SROWS = _NBLK * 32

_mesh = plsc.VectorSubcoreMesh(core_axis_name="c", subcore_axis_name="s")
_params = pltpu.CompilerParams(use_tc_tiling_on_sc=True,
                               needs_layout_passes=False)


def _wid():
    return lax.axis_index("s") * 2 + lax.axis_index("c")


@functools.partial(
    pl.kernel,
    mesh=_mesh,
    out_type=jax.ShapeDtypeStruct((_SROWS, 128), jnp.float32),
    scratch_types=[
        pltpu.VMEM((32, 128), jnp.float32),
        pltpu.VMEM((32, 128), jnp.float32),
        pltpu.SemaphoreType.DMA,
        pltpu.SemaphoreType.DMA,
    ],
    compiler_params=_params,
)
def _transpose_kernel(tab_t, scratch, buf_in, buf_out, isem, osem):
    w = _wid()
    lanes = lax.iota(jnp.int32, 16)

    def do_block(t):
        col = pl.multiple_of(t * 128, 128)
        pltpu.async_copy(
            tab_t.at[:, pl.ds(col, 128)], buf_in, isem).wait()
        for v0 in range(0, 128, 16):
            flat_base = (v0 + lanes) * _EMB
            for c in range(_EMB):
                flat = flat_base + c
                val = buf_in[c, pl.ds(v0, 16)]
                plsc.store_scatter(
                    buf_out,
                    [lax.shift_right_logical(flat, 7),
                     lax.bitwise_and(flat, 127)],
                    val)
        pltpu.async_copy(
            buf_out, scratch.at[pl.ds(pl.multiple_of(t * 32, 32), 32)],
            osem).wait()

    def body(k, carry):
        t = w + k * _NW

        @pl.when(t < _NBLK)
        def _():
            do_block(t)
        return carry

    lax.fori_loop(0, (_NBLK + _NW - 1) // _NW, body, 0)


@functools.partial(
    pl.kernel,
    mesh=_mesh,
    out_type=jax.ShapeDtypeStruct((_L, _EMB, _B), jnp.float32),
    scratch_types=[
        pltpu.VMEM((_L, 128), jnp.int32),
        pltpu.VMEM((128,), jnp.int32),
        pltpu.VMEM((128,), jnp.int32),
        pltpu.VMEM((128, 128), jnp.float32),
        pltpu.VMEM((_EMB, 128), jnp.float32),
        pltpu.SemaphoreType.DMA,
        pltpu.SemaphoreType.DMA,
        pltpu.SemaphoreType.DMA,
    ],
    compiler_params=_params,
)
def _gather_kernel(ids_t, scratch, out, idsv, rowbuf, offbuf, gbuf, obuf,
                   dsem, gsem, ssem):
    w = _wid()
    b0 = pl.multiple_of(w * 128, 128)
    pltpu.async_copy(ids_t.at[:, pl.ds(b0, 128)], idsv, dsem).wait()
    lanes = lax.iota(jnp.int32, 16)

    def do_row(l):
        for j0 in range(0, 128, 16):
            idv = idsv[l, pl.ds(j0, 16)]
            rowbuf[pl.ds(j0, 16)] = lax.shift_right_logical(idv, 2)
            offbuf[pl.ds(j0, 16)] = lax.bitwise_and(idv, 3) * _EMB
        pltpu.async_copy(scratch.at[rowbuf], gbuf, gsem).wait()
        for j0 in range(0, 128, 16):
            jvec = j0 + lanes
            offv = offbuf[pl.ds(j0, 16)]
            for c in range(_EMB):
                obuf[c, pl.ds(j0, 16)] = plsc.load_gather(
                    gbuf, [jvec, offv + c])
        pltpu.async_copy(obuf, out.at[l, :, pl.ds(b0, 128)], ssem).wait()

    def body(l, carry):
        do_row(l)
        return carry

    lax.fori_loop(0, _L, body, 0)


def kernel(input_ids, table):
    scratch = _transpose_kernel(table.T)
    out = _gather_kernel(input_ids.T, scratch)
    return jnp.transpose(out, (2, 0, 1))


# speedup vs baseline: 1.2626x; 1.2626x over previous
import functools

import jax
import jax.numpy as jnp
from jax import lax
from jax.experimental import pallas as pl
from jax.experimental.pallas import tpu as pltpu
from jax.experimental.pallas import tpu_sc as plsc

_EMB = 32
_B = 4096
_L = 200
_NW = 32
_VPAD = 1000064
_NBLK = _VPAD // 128
_SROWS = _NBLK * 32
_UPW = 61

_mesh = plsc.VectorSubcoreMesh(core_axis_name="c", subcore_axis_name="s")
_params = pltpu.CompilerParams(use_tc_tiling_on_sc=True,
                               needs_layout_passes=False)


def _wid():
    return lax.axis_index("s") * 2 + lax.axis_index("c")


@functools.partial(
    pl.kernel,
    mesh=_mesh,
    out_type=jax.ShapeDtypeStruct((_SROWS, 128), jnp.float32),
    scratch_types=[
        pltpu.VMEM((128, 128), jnp.float32),
        pltpu.VMEM((128, 128), jnp.float32),
        pltpu.VMEM((128, 128), jnp.float32),
        pltpu.VMEM((128, 128), jnp.float32),
        pltpu.SemaphoreType.DMA,
        pltpu.SemaphoreType.DMA,
        pltpu.SemaphoreType.DMA,
        pltpu.SemaphoreType.DMA,
    ],
    compiler_params=_params,
)
def _transpose_kernel(tab_t, scratch, in0, in1, ob0, ob1,
                      isem0, isem1, osem0, osem1):
    w = _wid()
    u0 = w * _UPW
    lanes = lax.iota(jnp.int32, 16)
    cpat = (lanes, lanes + 16)
    ins = (in0, in1)
    obs = (ob0, ob1)
    isem = (isem0, isem1)
    osem = (osem0, osem1)

    def in_dma(u, s):
        col = pl.multiple_of(u * 512, 128)
        for bb in range(4):
            pltpu.async_copy(
                tab_t.at[:, pl.ds(col + 128 * bb, 128)],
                ins[s].at[pl.ds(32 * bb, 32)], isem[s])

    def wait_in(s):
        for bb in range(4):
            pltpu.make_async_copy(
                tab_t.at[:, pl.ds(0, 128)],
                ins[s].at[pl.ds(0, 32)], isem[s]).wait()

    def wait_out(s):
        pltpu.make_async_copy(
            obs[s], scratch.at[pl.ds(0, 128)], osem[s]).wait()

    def transpose_unit(s):
        inb = ins[s]
        obb = obs[s]

        def rr_body(rr, carry):
            vbase = rr * 4
            for bb in range(4):
                r = bb * 32 + rr
                for q0 in range(0, 128, 16):
                    rowv = cpat[(q0 // 16) % 2] + (32 * bb)
                    lanev = jnp.full((16,), 0, jnp.int32) + (vbase + q0 // 32)
                    obb[r, pl.ds(q0, 16)] = plsc.load_gather(
                        inb, [rowv, lanev])
            return carry

        lax.fori_loop(0, 32, rr_body, 0)

    def store_out(u, s):
        pltpu.async_copy(
            obs[s], scratch.at[pl.ds(pl.multiple_of(u * 128, 128), 128)],
            osem[s])

    in_dma(u0, 0)

    def body(p, carry):
        ua = u0 + 2 * p
        in_dma(ua + 1, 1)
        wait_in(0)

        @pl.when(p >= 1)
        def _():
            wait_out(0)
        transpose_unit(0)
        store_out(ua, 0)

        in_dma(ua + 2, 0)
        wait_in(1)

        @pl.when(p >= 1)
        def _():
            wait_out(1)
        transpose_unit(1)
        store_out(ua + 1, 1)
        return carry

    lax.fori_loop(0, (_UPW - 1) // 2, body, 0)

    wait_in(0)
    wait_out(0)
    transpose_unit(0)
    store_out(u0 + _UPW - 1, 0)
    wait_out(0)
    wait_out(1)

    @pl.when(w < 5)
    def _():
        blk = 7808 + w
        col = pl.multiple_of(blk * 128, 128)
        pltpu.async_copy(
            tab_t.at[:, pl.ds(col, 128)],
            in1.at[pl.ds(0, 32)], isem1).wait()

        def rr_body(rr, carry):
            vbase = rr * 4
            for q0 in range(0, 128, 16):
                rowv = cpat[(q0 // 16) % 2]
                lanev = jnp.full((16,), 0, jnp.int32) + (vbase + q0 // 32)
                ob1[rr, pl.ds(q0, 16)] = plsc.load_gather(
                    in1.at[pl.ds(0, 32)], [rowv, lanev])
            return carry

        lax.fori_loop(0, 32, rr_body, 0)
        pltpu.async_copy(
            ob1.at[pl.ds(0, 32)],
            scratch.at[pl.ds(pl.multiple_of(blk * 32, 32), 32)],
            osem1).wait()


@functools.partial(
    pl.kernel,
    mesh=_mesh,
    out_type=jax.ShapeDtypeStruct((_L, _EMB, _B), jnp.float32),
    scratch_types=[
        pltpu.VMEM((_L, 128), jnp.int32),
        pltpu.VMEM((128,), jnp.int32),
        pltpu.VMEM((128,), jnp.int32),
        pltpu.VMEM((128, 128), jnp.float32),
        pltpu.VMEM((128, 128), jnp.float32),
        pltpu.VMEM((_EMB, 128), jnp.float32),
        pltpu.VMEM((_EMB, 128), jnp.float32),
        pltpu.SemaphoreType.DMA,
        pltpu.SemaphoreType.DMA,
        pltpu.SemaphoreType.DMA,
        pltpu.SemaphoreType.DMA,
        pltpu.SemaphoreType.DMA,
    ],
    compiler_params=_params,
)
def _gather_kernel(ids_t, scratch, out, idsv, rb0, rb1, g0, g1, ob0, ob1,
                   dsem, gsem0, gsem1, ssem0, ssem1):
    w = _wid()
    b0 = pl.multiple_of(w * 128, 128)
    pltpu.async_copy(ids_t.at[:, pl.ds(b0, 128)], idsv, dsem).wait()
    lanes = lax.iota(jnp.int32, 16)
    jvecs = tuple(lanes + j0 for j0 in range(0, 128, 16))
    rbs = (rb0, rb1)
    gs = (g0, g1)
    obs = (ob0, ob1)
    gsem = (gsem0, gsem1)
    ssem = (ssem0, ssem1)

    def prep_and_fire(l, s):
        offs = []
        for c8 in range(8):
            idv = idsv[l, pl.ds(c8 * 16, 16)]
            rbs[s][pl.ds(c8 * 16, 16)] = lax.shift_right_logical(idv, 2)
            offs.append(lax.shift_left(lax.bitwise_and(idv, 3), 5))
        copy = pltpu.async_copy(scratch.at[rbs[s]], gs[s], gsem[s])
        return tuple(offs)

    def wait_g(s):
        pltpu.make_async_copy(scratch.at[rbs[s]], gs[s], gsem[s]).wait()

    def wait_s(s):
        pltpu.make_async_copy(
            obs[s], out.at[0, :, pl.ds(b0, 128)], ssem[s]).wait()

    def assemble_and_store(l, offs, s):
        for c8 in range(8):
            offv = offs[c8]
            jv = jvecs[c8]
            for c in range(_EMB):
                obs[s][c, pl.ds(c8 * 16, 16)] = plsc.load_gather(
                    gs[s], [jv, offv + c])
        pltpu.async_copy(obs[s], out.at[l, :, pl.ds(b0, 128)], ssem[s])

    offs_a = prep_and_fire(0, 0)

    def body(p, offs_cur):
        l0 = 2 * p
        offs_n1 = prep_and_fire(l0 + 1, 1)
        wait_g(0)

        @pl.when(p >= 1)
        def _():
            wait_s(0)
        assemble_and_store(l0, offs_cur, 0)

        lnxt = jnp.minimum(l0 + 2, _L - 1)
        offs_n2 = prep_and_fire(lnxt, 0)
        wait_g(1)

        @pl.when(p >= 1)
        def _():
            wait_s(1)
        assemble_and_store(l0 + 1, offs_n1, 1)
        return offs_n2

    lax.fori_loop(0, _L // 2, body, offs_a)
    wait_g(0)
    wait_s(0)
    wait_s(1)


def kernel(input_ids, table):
    scratch = _transpose_kernel(table.T)
    out = _gather_kernel(input_ids.T, scratch)
    return jnp.transpose(out, (2, 0, 1))


# speedup vs baseline: 1.8097x; 1.4333x over previous
import functools

import jax
import jax.numpy as jnp
from jax import lax
from jax.experimental import pallas as pl
from jax.experimental.pallas import tpu as pltpu
from jax.experimental.pallas import tpu_sc as plsc

_EMB = 32
_B = 4096
_L = 200
_NW = 32
_VPAD = 1000064
_NBLK = _VPAD // 128
_SROWS = _NBLK * 32
_UPW = 61

_mesh = plsc.VectorSubcoreMesh(core_axis_name="c", subcore_axis_name="s")
_params = pltpu.CompilerParams(use_tc_tiling_on_sc=True,
                               needs_layout_passes=False)


def _wid():
    return lax.axis_index("s") * 2 + lax.axis_index("c")


@functools.partial(
    pl.kernel,
    mesh=_mesh,
    out_type=jax.ShapeDtypeStruct((_SROWS, 128), jnp.float32),
    scratch_types=[
        pltpu.VMEM((128, 128), jnp.float32),
        pltpu.VMEM((128, 128), jnp.float32),
        pltpu.VMEM((128, 128), jnp.float32),
        pltpu.VMEM((128, 128), jnp.float32),
        pltpu.SemaphoreType.DMA,
        pltpu.SemaphoreType.DMA,
        pltpu.SemaphoreType.DMA,
        pltpu.SemaphoreType.DMA,
    ],
    compiler_params=_params,
)
def _transpose_kernel(tab_t, scratch, in0, in1, ob0, ob1,
                      isem0, isem1, osem0, osem1):
    w = _wid()
    u0 = w * _UPW
    lanes = lax.iota(jnp.int32, 16)
    rowpat = tuple(
        lax.shift_right_logical(lanes + vv0, 2) for vv0 in range(0, 128, 16))
    lpat = tuple(
        lax.shift_left(lax.bitwise_and(lanes + vv0, 3), 5)
        for vv0 in range(0, 128, 16))
    ins = (in0, in1)
    obs = (ob0, ob1)
    isem = (isem0, isem1)
    osem = (osem0, osem1)

    def in_dma(u, s):
        col = pl.multiple_of(u * 512, 128)
        for bb in range(4):
            pltpu.async_copy(
                tab_t.at[:, pl.ds(col + 128 * bb, 128)],
                ins[s].at[pl.ds(32 * bb, 32)], isem[s])

    def wait_in(s):
        for bb in range(4):
            pltpu.make_async_copy(
                tab_t.at[:, pl.ds(0, 128)],
                ins[s].at[pl.ds(0, 32)], isem[s]).wait()

    def wait_out(s):
        pltpu.make_async_copy(
            obs[s], scratch.at[pl.ds(0, 128)], osem[s]).wait()

    def transpose_rows(src, dst, nbb):
        def c_body(c, carry):
            for bb in range(nbb):
                vals = [src[bb * 32 + c, pl.ds(vv0, 16)]
                        for vv0 in range(0, 128, 16)]
                for k8 in range(8):
                    plsc.store_scatter(
                        dst, [rowpat[k8] + (32 * bb), lpat[k8] + c],
                        vals[k8])
            return carry

        lax.fori_loop(0, 32, c_body, 0)

    def store_out(u, s):
        pltpu.async_copy(
            obs[s], scratch.at[pl.ds(pl.multiple_of(u * 128, 128), 128)],
            osem[s])

    in_dma(u0, 0)

    def body(p, carry):
        ua = u0 + 2 * p
        in_dma(ua + 1, 1)
        wait_in(0)

        @pl.when(p >= 1)
        def _():
            wait_out(0)
        transpose_rows(ins[0], obs[0], 4)
        store_out(ua, 0)

        in_dma(ua + 2, 0)
        wait_in(1)

        @pl.when(p >= 1)
        def _():
            wait_out(1)
        transpose_rows(ins[1], obs[1], 4)
        store_out(ua + 1, 1)
        return carry

    lax.fori_loop(0, (_UPW - 1) // 2, body, 0)

    wait_in(0)
    wait_out(0)
    transpose_rows(ins[0], obs[0], 4)
    store_out(u0 + _UPW - 1, 0)
    wait_out(0)
    wait_out(1)

    @pl.when(w < 5)
    def _():
        blk = 7808 + w
        col = pl.multiple_of(blk * 128, 128)
        pltpu.async_copy(
            tab_t.at[:, pl.ds(col, 128)],
            in1.at[pl.ds(0, 32)], isem1).wait()
        transpose_rows(in1, ob1, 1)
        pltpu.async_copy(
            ob1.at[pl.ds(0, 32)],
            scratch.at[pl.ds(pl.multiple_of(blk * 32, 32), 32)],
            osem1).wait()


@functools.partial(
    pl.kernel,
    mesh=_mesh,
    out_type=jax.ShapeDtypeStruct((_L, _EMB, _B), jnp.float32),
    scratch_types=[
        pltpu.VMEM((_L, 128), jnp.int32),
        pltpu.VMEM((128,), jnp.int32),
        pltpu.VMEM((128,), jnp.int32),
        pltpu.VMEM((128, 128), jnp.float32),
        pltpu.VMEM((128, 128), jnp.float32),
        pltpu.VMEM((_EMB, 128), jnp.float32),
        pltpu.VMEM((_EMB, 128), jnp.float32),
        pltpu.SemaphoreType.DMA,
        pltpu.SemaphoreType.DMA,
        pltpu.SemaphoreType.DMA,
        pltpu.SemaphoreType.DMA,
        pltpu.SemaphoreType.DMA,
    ],
    compiler_params=_params,
)
def _gather_kernel(ids_t, scratch, out, idsv, rb0, rb1, g0, g1, ob0, ob1,
                   dsem, gsem0, gsem1, ssem0, ssem1):
    w = _wid()
    b0 = pl.multiple_of(w * 128, 128)
    pltpu.async_copy(ids_t.at[:, pl.ds(b0, 128)], idsv, dsem).wait()
    lanes = lax.iota(jnp.int32, 16)
    jvecs = tuple(lanes + j0 for j0 in range(0, 128, 16))
    rbs = (rb0, rb1)
    gs = (g0, g1)
    obs = (ob0, ob1)
    gsem = (gsem0, gsem1)
    ssem = (ssem0, ssem1)

    def prep_and_fire(l, s):
        offs = []
        for c8 in range(8):
            idv = idsv[l, pl.ds(c8 * 16, 16)]
            rbs[s][pl.ds(c8 * 16, 16)] = lax.shift_right_logical(idv, 2)
            offs.append(lax.shift_left(lax.bitwise_and(idv, 3), 5))
        pltpu.async_copy(scratch.at[rbs[s]], gs[s], gsem[s])
        return tuple(offs)

    def wait_g(s):
        pltpu.make_async_copy(scratch.at[rbs[s]], gs[s], gsem[s]).wait()

    def wait_s(s):
        pltpu.make_async_copy(
            obs[s], out.at[0, :, pl.ds(b0, 128)], ssem[s]).wait()

    def assemble_and_store(l, offs, s):
        for c8 in range(8):
            offv = offs[c8]
            jv = jvecs[c8]
            for cw in range(0, _EMB, 8):
                vals = [plsc.load_gather(gs[s], [jv, offv + c])
                        for c in range(cw, cw + 8)]
                for k, c in enumerate(range(cw, cw + 8)):
                    obs[s][c, pl.ds(c8 * 16, 16)] = vals[k]
        pltpu.async_copy(obs[s], out.at[l, :, pl.ds(b0, 128)], ssem[s])

    offs_a = prep_and_fire(0, 0)

    def body(p, offs_cur):
        l0 = 2 * p
        offs_n1 = prep_and_fire(l0 + 1, 1)
        wait_g(0)

        @pl.when(p >= 1)
        def _():
            wait_s(0)
        assemble_and_store(l0, offs_cur, 0)

        offs_n2 = prep_and_fire(jnp.minimum(l0 + 2, _L - 1), 0)
        wait_g(1)

        @pl.when(p >= 1)
        def _():
            wait_s(1)
        assemble_and_store(l0 + 1, offs_n1, 1)
        return offs_n2

    lax.fori_loop(0, _L // 2, body, offs_a)
    wait_g(0)
    wait_s(0)
    wait_s(1)


def kernel(input_ids, table):
    scratch = _transpose_kernel(table.T)
    out = _gather_kernel(input_ids.T, scratch)
    return jnp.transpose(out, (2, 0, 1))


# speedup vs baseline: 2.4497x; 1.3537x over previous
import functools

import jax
import jax.numpy as jnp
from jax import lax
from jax.experimental import pallas as pl
from jax.experimental.pallas import tpu as pltpu
from jax.experimental.pallas import tpu_sc as plsc

_EMB = 32
_B = 4096
_L = 200
_NW = 32
_VPAD = 1000064
_NBLK = _VPAD // 128
_SROWS = _NBLK * 32
_UPW = 61

_mesh = plsc.VectorSubcoreMesh(core_axis_name="c", subcore_axis_name="s")
_params = pltpu.CompilerParams(use_tc_tiling_on_sc=True,
                               needs_layout_passes=False)


def _wid():
    return lax.axis_index("s") * 2 + lax.axis_index("c")


@functools.partial(
    pl.kernel,
    mesh=_mesh,
    out_type=jax.ShapeDtypeStruct((_SROWS, 128), jnp.float32),
    scratch_types=[
        pltpu.VMEM((128, 128), jnp.float32),
        pltpu.VMEM((128, 128), jnp.float32),
        pltpu.VMEM((128, 128), jnp.float32),
        pltpu.VMEM((128, 128), jnp.float32),
        pltpu.SemaphoreType.DMA,
        pltpu.SemaphoreType.DMA,
        pltpu.SemaphoreType.DMA,
        pltpu.SemaphoreType.DMA,
    ],
    compiler_params=_params,
)
def _transpose_kernel(tab_t, scratch, in0, in1, ob0, ob1,
                      isem0, isem1, osem0, osem1):
    w = _wid()
    u0 = w * _UPW
    lanes = lax.iota(jnp.int32, 16)
    diag = tuple(lax.bitwise_and(lanes + k, 15) for k in range(16))
    dstlane0 = lax.shift_left(lax.bitwise_and(lanes, 3), 5)
    dstrow0 = lax.shift_right_logical(lanes, 2)
    ins = (in0, in1)
    obs = (ob0, ob1)
    isem = (isem0, isem1)
    osem = (osem0, osem1)

    def in_dma(u, s):
        col = pl.multiple_of(u * 512, 128)
        for bb in range(4):
            pltpu.async_copy(
                tab_t.at[:, pl.ds(col + 128 * bb, 128)],
                ins[s].at[pl.ds(32 * bb, 32)], isem[s])

    def wait_in(s):
        for bb in range(4):
            pltpu.make_async_copy(
                tab_t.at[:, pl.ds(0, 128)],
                ins[s].at[pl.ds(0, 32)], isem[s]).wait()

    def wait_out(s):
        pltpu.make_async_copy(
            obs[s], scratch.at[pl.ds(0, 128)], osem[s]).wait()

    def transpose_rows(src, dst, nbb):
        def vo_body(vo, carry):
            colv = lanes + vo * 16
            for bb in range(nbb):
                rowd = dstrow0 + (vo * 4 + 32 * bb)
                for ch in range(2):
                    for k in range(16):
                        rowv = diag[k] + (32 * bb + 16 * ch)
                        lanev = dstlane0 + (diag[k] + 16 * ch)
                        val = plsc.load_gather(src, [rowv, colv])
                        plsc.store_scatter(dst, [rowd, lanev], val)
            return carry

        lax.fori_loop(0, 8, vo_body, 0)

    def store_out(u, s):
        pltpu.async_copy(
            obs[s], scratch.at[pl.ds(pl.multiple_of(u * 128, 128), 128)],
            osem[s])

    in_dma(u0, 0)

    def body(p, carry):
        ua = u0 + 2 * p
        in_dma(ua + 1, 1)
        wait_in(0)

        @pl.when(p >= 1)
        def _():
            wait_out(0)
        transpose_rows(ins[0], obs[0], 4)
        store_out(ua, 0)

        in_dma(ua + 2, 0)
        wait_in(1)

        @pl.when(p >= 1)
        def _():
            wait_out(1)
        transpose_rows(ins[1], obs[1], 4)
        store_out(ua + 1, 1)
        return carry

    lax.fori_loop(0, (_UPW - 1) // 2, body, 0)

    wait_in(0)
    wait_out(0)
    transpose_rows(ins[0], obs[0], 4)
    store_out(u0 + _UPW - 1, 0)
    wait_out(0)
    wait_out(1)

    @pl.when(w < 5)
    def _():
        blk = 7808 + w
        col = pl.multiple_of(blk * 128, 128)
        pltpu.async_copy(
            tab_t.at[:, pl.ds(col, 128)],
            in1.at[pl.ds(0, 32)], isem1).wait()
        transpose_rows(in1, ob1, 1)
        pltpu.async_copy(
            ob1.at[pl.ds(0, 32)],
            scratch.at[pl.ds(pl.multiple_of(blk * 32, 32), 32)],
            osem1).wait()


@functools.partial(
    pl.kernel,
    mesh=_mesh,
    out_type=jax.ShapeDtypeStruct((_L, _EMB, _B), jnp.float32),
    scratch_types=[
        pltpu.VMEM((_L, 128), jnp.int32),
        pltpu.VMEM((128,), jnp.int32),
        pltpu.VMEM((128,), jnp.int32),
        pltpu.VMEM((128, 128), jnp.float32),
        pltpu.VMEM((128, 128), jnp.float32),
        pltpu.VMEM((_EMB, 128), jnp.float32),
        pltpu.VMEM((_EMB, 128), jnp.float32),
        pltpu.SemaphoreType.DMA,
        pltpu.SemaphoreType.DMA,
        pltpu.SemaphoreType.DMA,
        pltpu.SemaphoreType.DMA,
        pltpu.SemaphoreType.DMA,
    ],
    compiler_params=_params,
)
def _gather_kernel(ids_t, scratch, out, idsv, rb0, rb1, g0, g1, ob0, ob1,
                   dsem, gsem0, gsem1, ssem0, ssem1):
    w = _wid()
    b0 = pl.multiple_of(w * 128, 128)
    pltpu.async_copy(ids_t.at[:, pl.ds(b0, 128)], idsv, dsem).wait()
    lanes = lax.iota(jnp.int32, 16)
    jvecs = tuple(lanes + j0 for j0 in range(0, 128, 16))
    rbs = (rb0, rb1)
    gs = (g0, g1)
    obs = (ob0, ob1)
    gsem = (gsem0, gsem1)
    ssem = (ssem0, ssem1)

    def prep_and_fire(l, s):
        offs = []
        for c8 in range(8):
            idv = idsv[l, pl.ds(c8 * 16, 16)]
            rbs[s][pl.ds(c8 * 16, 16)] = lax.shift_right_logical(idv, 2)
            offs.append(lax.shift_left(lax.bitwise_and(idv, 3), 5))
        pltpu.async_copy(scratch.at[rbs[s]], gs[s], gsem[s])
        return tuple(offs)

    def wait_g(s):
        pltpu.make_async_copy(scratch.at[rbs[s]], gs[s], gsem[s]).wait()

    def wait_s(s):
        pltpu.make_async_copy(
            obs[s], out.at[0, :, pl.ds(b0, 128)], ssem[s]).wait()

    def assemble_and_store(l, offs, s):
        for c8 in range(8):
            offv = offs[c8]
            jv = jvecs[c8]
            for cw in range(0, _EMB, 8):
                vals = [plsc.load_gather(gs[s], [jv, offv + c])
                        for c in range(cw, cw + 8)]
                for k, c in enumerate(range(cw, cw + 8)):
                    obs[s][c, pl.ds(c8 * 16, 16)] = vals[k]
        pltpu.async_copy(obs[s], out.at[l, :, pl.ds(b0, 128)], ssem[s])

    offs_a = prep_and_fire(0, 0)

    def body(p, offs_cur):
        l0 = 2 * p
        offs_n1 = prep_and_fire(l0 + 1, 1)
        wait_g(0)

        @pl.when(p >= 1)
        def _():
            wait_s(0)
        assemble_and_store(l0, offs_cur, 0)

        offs_n2 = prep_and_fire(jnp.minimum(l0 + 2, _L - 1), 0)
        wait_g(1)

        @pl.when(p >= 1)
        def _():
            wait_s(1)
        assemble_and_store(l0 + 1, offs_n1, 1)
        return offs_n2

    lax.fori_loop(0, _L // 2, body, offs_a)
    wait_g(0)
    wait_s(0)
    wait_s(1)


def kernel(input_ids, table):
    scratch = _transpose_kernel(table.T)
    out = _gather_kernel(input_ids.T, scratch)
    return jnp.transpose(out, (2, 0, 1))


# speedup vs baseline: 2.8629x; 1.1687x over previous
import functools

import jax
import jax.numpy as jnp
from jax import lax
from jax.experimental import pallas as pl
from jax.experimental.pallas import tpu as pltpu
from jax.experimental.pallas import tpu_sc as plsc

_EMB = 32
_B = 4096
_L = 200
_NW = 32
_VPAD = 1000064
_NBLK = _VPAD // 128
_SROWS = _NBLK * 32
_UPW = 61

_mesh = plsc.VectorSubcoreMesh(core_axis_name="c", subcore_axis_name="s")
_params = pltpu.CompilerParams(use_tc_tiling_on_sc=True,
                               needs_layout_passes=False)


def _wid():
    return lax.axis_index("s") * 2 + lax.axis_index("c")


@functools.partial(
    pl.kernel,
    mesh=_mesh,
    out_type=jax.ShapeDtypeStruct((_SROWS, 128), jnp.float32),
    scratch_types=[
        pltpu.VMEM((128, 128), jnp.float32),
        pltpu.VMEM((128, 128), jnp.float32),
        pltpu.VMEM((128, 128), jnp.float32),
        pltpu.VMEM((128, 128), jnp.float32),
        pltpu.SemaphoreType.DMA,
        pltpu.SemaphoreType.DMA,
        pltpu.SemaphoreType.DMA,
        pltpu.SemaphoreType.DMA,
    ],
    compiler_params=_params,
)
def _transpose_kernel(tab_t, scratch, in0, in1, ob0, ob1,
                      isem0, isem1, osem0, osem1):
    w = _wid()
    u0 = w * _UPW
    lanes = lax.iota(jnp.int32, 16)
    diag = tuple(lax.bitwise_and(lanes + k, 15) for k in range(16))
    dstlane0 = lax.shift_left(lax.bitwise_and(lanes, 3), 5)
    dstrow0 = lax.shift_right_logical(lanes, 2)
    ins = (in0, in1)
    obs = (ob0, ob1)
    isem = (isem0, isem1)
    osem = (osem0, osem1)

    def in_dma(u, s):
        col = pl.multiple_of(u * 512, 128)
        for bb in range(4):
            pltpu.async_copy(
                tab_t.at[:, pl.ds(col + 128 * bb, 128)],
                ins[s].at[pl.ds(32 * bb, 32)], isem[s])

    def wait_in(s):
        for bb in range(4):
            pltpu.make_async_copy(
                tab_t.at[:, pl.ds(0, 128)],
                ins[s].at[pl.ds(0, 32)], isem[s]).wait()

    def wait_out(s):
        pltpu.make_async_copy(
            obs[s], scratch.at[pl.ds(0, 128)], osem[s]).wait()

    def transpose_rows(src, dst, nbb):
        def vo_body(vo, carry):
            colv = lanes + vo * 16
            for bb in range(nbb):
                rowd = dstrow0 + (vo * 4 + 32 * bb)
                for ch in range(2):
                    for k in range(16):
                        rowv = diag[k] + (32 * bb + 16 * ch)
                        lanev = dstlane0 + (diag[k] + 16 * ch)
                        val = plsc.load_gather(src, [rowv, colv])
                        plsc.store_scatter(dst, [rowd, lanev], val)
            return carry

        lax.fori_loop(0, 8, vo_body, 0)

    def store_out(u, s):
        pltpu.async_copy(
            obs[s], scratch.at[pl.ds(pl.multiple_of(u * 128, 128), 128)],
            osem[s])

    in_dma(u0, 0)

    def body(p, carry):
        ua = u0 + 2 * p
        in_dma(ua + 1, 1)
        wait_in(0)

        @pl.when(p >= 1)
        def _():
            wait_out(0)
        transpose_rows(ins[0], obs[0], 4)
        store_out(ua, 0)

        in_dma(ua + 2, 0)
        wait_in(1)

        @pl.when(p >= 1)
        def _():
            wait_out(1)
        transpose_rows(ins[1], obs[1], 4)
        store_out(ua + 1, 1)
        return carry

    lax.fori_loop(0, (_UPW - 1) // 2, body, 0)

    wait_in(0)
    wait_out(0)
    transpose_rows(ins[0], obs[0], 4)
    store_out(u0 + _UPW - 1, 0)
    wait_out(0)
    wait_out(1)

    @pl.when(w < 5)
    def _():
        blk = 7808 + w
        col = pl.multiple_of(blk * 128, 128)
        pltpu.async_copy(
            tab_t.at[:, pl.ds(col, 128)],
            in1.at[pl.ds(0, 32)], isem1).wait()
        transpose_rows(in1, ob1, 1)
        pltpu.async_copy(
            ob1.at[pl.ds(0, 32)],
            scratch.at[pl.ds(pl.multiple_of(blk * 32, 32), 32)],
            osem1).wait()


@functools.partial(
    pl.kernel,
    mesh=_mesh,
    out_type=jax.ShapeDtypeStruct((_L, _EMB, _B), jnp.float32),
    scratch_types=[
        pltpu.VMEM((_L, 128), jnp.int32),
        pltpu.VMEM((128,), jnp.int32),
        pltpu.VMEM((128,), jnp.int32),
        pltpu.VMEM((128,), jnp.int32),
        pltpu.VMEM((128,), jnp.int32),
        pltpu.VMEM((128, 128), jnp.float32),
        pltpu.VMEM((128, 128), jnp.float32),
        pltpu.VMEM((_EMB, 128), jnp.float32),
        pltpu.VMEM((_EMB, 128), jnp.float32),
        pltpu.SemaphoreType.DMA,
        pltpu.SemaphoreType.DMA,
        pltpu.SemaphoreType.DMA,
        pltpu.SemaphoreType.DMA,
        pltpu.SemaphoreType.DMA,
    ],
    compiler_params=_params,
)
def _gather_kernel(ids_t, scratch, out, idsv, rb0, rb1, of0, of1,
                   g0, g1, ob0, ob1, dsem, gsem0, gsem1, ssem0, ssem1):
    w = _wid()
    b0 = pl.multiple_of(w * 128, 128)
    pltpu.async_copy(ids_t.at[:, pl.ds(b0, 128)], idsv, dsem).wait()
    lanes = lax.iota(jnp.int32, 16)
    jvecs = tuple(lanes + j0 for j0 in range(0, 128, 16))
    diag = tuple(lax.bitwise_and(lanes + k, 15) for k in range(16))
    rbs = (rb0, rb1)
    ofs = (of0, of1)
    gs = (g0, g1)
    obs = (ob0, ob1)
    gsem = (gsem0, gsem1)
    ssem = (ssem0, ssem1)

    def prep_and_fire(l, s):
        for c8 in range(8):
            idv = idsv[l, pl.ds(c8 * 16, 16)]
            rbs[s][pl.ds(c8 * 16, 16)] = lax.shift_right_logical(idv, 2)
            ofs[s][pl.ds(c8 * 16, 16)] = lax.shift_left(
                lax.bitwise_and(idv, 3), 5)
        pltpu.async_copy(scratch.at[rbs[s]], gs[s], gsem[s])

    def wait_g(s):
        pltpu.make_async_copy(scratch.at[rbs[s]], gs[s], gsem[s]).wait()

    def wait_s(s):
        pltpu.make_async_copy(
            obs[s], out.at[0, :, pl.ds(b0, 128)], ssem[s]).wait()

    def assemble_and_store(l, s):
        def c8_body(c8, carry):
            jv = lanes + c8 * 16
            offv = ofs[s][pl.ds(c8 * 16, 16)]
            for ch in range(2):
                offc = offv + (16 * ch)
                for k in range(16):
                    val = plsc.load_gather(gs[s], [jv, offc + diag[k]])
                    plsc.store_scatter(
                        obs[s], [diag[k] + (16 * ch), jv], val)
            return carry

        lax.fori_loop(0, 8, c8_body, 0)
        pltpu.async_copy(obs[s], out.at[l, :, pl.ds(b0, 128)], ssem[s])

    prep_and_fire(0, 0)

    def body(p, carry):
        l0 = 2 * p
        prep_and_fire(l0 + 1, 1)
        wait_g(0)

        @pl.when(p >= 1)
        def _():
            wait_s(0)
        assemble_and_store(l0, 0)

        prep_and_fire(jnp.minimum(l0 + 2, _L - 1), 0)
        wait_g(1)

        @pl.when(p >= 1)
        def _():
            wait_s(1)
        assemble_and_store(l0 + 1, 1)
        return carry

    lax.fori_loop(0, _L // 2, body, 0)
    wait_g(0)
    wait_s(0)
    wait_s(1)


def kernel(input_ids, table):
    scratch = _transpose_kernel(table.T)
    out = _gather_kernel(input_ids.T, scratch)
    return jnp.transpose(out, (2, 0, 1))
